# baseline (speedup 1.0000x reference)
import jax
import jax.numpy as jnp
from jax import lax
from jax.experimental import pallas as pl
from jax.experimental.pallas import tpu as pltpu
from jax.experimental.pallas import tpu_sc as plsc

N_NODES = 10000
D = 128
E = 320000
NC = 2
NS = 16
NW = NC * NS
EPW = E // NW
B = 80
NCHUNK = EPW // B
NJ = D // 16


def _sc_body(hs, hd, isrc, idst, w, out,
             idxs_v, idxd_v, w_v, out_v, s0, t0, s1, t1,
             is0, id0, is1, id1, m_v, sem0, sem1):
    c = lax.axis_index("c")
    s = lax.axis_index("s")
    wid = s * NC + c
    base = wid * EPW
    pltpu.sync_copy(isrc.at[pl.ds(base, EPW)], idxs_v)
    pltpu.sync_copy(idst.at[pl.ds(base, EPW)], idxd_v)
    pltpu.sync_copy(w, w_v)

    def start(i, sb, tb, isb, idb, sem):
        del i, sb, tb, isb, idb, sem

    def drain(sb, tb, isb, idb, sem):
        del sb, tb, isb, idb, sem

    iot16 = lax.iota(jnp.int32, 16) * 16

    def compute(i, sb, tb):
        def group(g, _):
            e0 = g * 16
            for e in range(16):
                acc = jnp.zeros((16,), jnp.float32)
                for j in range(NJ):
                    sj = sb[e0 + e, pl.ds(j * 16, 16)]
                    tj = tb[e0 + e, pl.ds(j * 16, 16)]
                    wj = w_v[pl.ds(j * 16, 16)]
                    acc = acc + sj * (tj * wj)
                m_v[pl.ds(e * 16, 16)] = acc
            r = jnp.zeros((16,), jnp.float32)
            for l in range(16):
                r = r + plsc.load_gather(m_v, [iot16 + l])
            out_v[pl.dslice(i * B + e0, 16)] = r
            return 0

        lax.fori_loop(0, B // 16, group, 0)

    start(0, s0, t0, is0, id0, sem0)

    def outer(k, _):
        i0 = 2 * k
        start(i0 + 1, s1, t1, is1, id1, sem1)
        drain(s0, t0, is0, id0, sem0)
        compute(i0, s0, t0)

        @pl.when(i0 + 2 < NCHUNK)
        def _():
            start(i0 + 2, s0, t0, is0, id0, sem0)

        drain(s1, t1, is1, id1, sem1)
        compute(i0 + 1, s1, t1)
        return 0

    lax.fori_loop(0, (NCHUNK - 1) // 2, outer, 0)
    drain(s0, t0, is0, id0, sem0)
    compute(NCHUNK - 1, s0, t0)

    pltpu.sync_copy(out_v, out.at[pl.ds(base, EPW)])


def kernel(h_src, h_dst, edge_label_index, W):
    w = W[0]
    isrc = edge_label_index[0].astype(jnp.int32)
    idst = edge_label_index[1].astype(jnp.int32)
    mesh = plsc.VectorSubcoreMesh(
        core_axis_name="c", subcore_axis_name="s", num_cores=NC, num_subcores=NS
    )
    fn = pl.kernel(
        _sc_body,
        out_type=jax.ShapeDtypeStruct((E,), jnp.float32),
        mesh=mesh,
        compiler_params=pltpu.CompilerParams(needs_layout_passes=False),
        scratch_types=[
            pltpu.VMEM((EPW,), jnp.int32),
            pltpu.VMEM((EPW,), jnp.int32),
            pltpu.VMEM((D,), jnp.float32),
            pltpu.VMEM((EPW,), jnp.float32),
            pltpu.VMEM((B, D), jnp.float32),
            pltpu.VMEM((B, D), jnp.float32),
            pltpu.VMEM((B, D), jnp.float32),
            pltpu.VMEM((B, D), jnp.float32),
            pltpu.VMEM((B,), jnp.int32),
            pltpu.VMEM((B,), jnp.int32),
            pltpu.VMEM((B,), jnp.int32),
            pltpu.VMEM((B,), jnp.int32),
            pltpu.VMEM((256,), jnp.float32),
            pltpu.SemaphoreType.DMA,
            pltpu.SemaphoreType.DMA,
        ],
    )
    return fn(h_src, h_dst, isrc, idst, w)


# speedup vs baseline: 1.2915x; 1.2915x over previous
import jax
import jax.numpy as jnp
from jax import lax
from jax.experimental import pallas as pl
from jax.experimental.pallas import tpu as pltpu
from jax.experimental.pallas import tpu_sc as plsc

N_NODES = 10000
D = 128
E = 320000
NC = 2
NS = 16
NW = NC * NS
EPW = E // NW
B = 80
NCHUNK = EPW // B
NJ = D // 16
ROWBLK = 1000


def _prescale_body(s_ref, w_ref, os_ref):
    os_ref[...] = s_ref[...] * w_ref[...]


def _prescale(h_src, w):
    return pl.pallas_call(
        _prescale_body,
        grid=(N_NODES // ROWBLK,),
        in_specs=[
            pl.BlockSpec((ROWBLK, D), lambda i: (i, 0)),
            pl.BlockSpec((1, D), lambda i: (0, 0)),
        ],
        out_specs=pl.BlockSpec((ROWBLK, D), lambda i: (i, 0)),
        out_shape=jax.ShapeDtypeStruct((N_NODES, D), jnp.float32),
    )(h_src, w.reshape(1, D))


def _sc_body(hs, hd, isrc, idst, out,
             idxs_v, idxd_v, out_v, s0, t0, s1, t1, m_v, sem0, sem1):
    c = lax.axis_index("c")
    s = lax.axis_index("s")
    wid = s * NC + c
    base = wid * EPW
    pltpu.sync_copy(isrc.at[pl.ds(base, EPW)], idxs_v)
    pltpu.sync_copy(idst.at[pl.ds(base, EPW)], idxd_v)

    def start(i, sb, tb, sem):
        pltpu.async_copy(hs.at[idxs_v.at[pl.ds(i * B, B)]], sb, sem)
        pltpu.async_copy(hd.at[idxd_v.at[pl.ds(i * B, B)]], tb, sem)

    def drain(sb, tb, sem):
        pltpu.make_async_copy(hs.at[idxs_v.at[pl.ds(0, B)]], sb, sem).wait()
        pltpu.make_async_copy(hd.at[idxd_v.at[pl.ds(0, B)]], tb, sem).wait()

    iot16 = lax.iota(jnp.int32, 16) * 16

    def compute(i, sb, tb):
        def group(g, _):
            e0 = g * 16
            for e in range(16):
                r = e0 + e
                acc0 = sb[r, pl.ds(0, 16)] * tb[r, pl.ds(0, 16)]
                acc1 = sb[r, pl.ds(16, 16)] * tb[r, pl.ds(16, 16)]
                for j in range(2, NJ, 2):
                    acc0 = acc0 + sb[r, pl.ds(j * 16, 16)] * tb[r, pl.ds(j * 16, 16)]
                    acc1 = acc1 + sb[r, pl.ds((j + 1) * 16, 16)] * tb[r, pl.ds((j + 1) * 16, 16)]
                m_v[pl.ds(e * 16, 16)] = acc0 + acc1
            cols = [plsc.load_gather(m_v, [iot16 + l]) for l in range(16)]
            while len(cols) > 1:
                cols = [a + b for a, b in zip(cols[::2], cols[1::2])]
            out_v[pl.dslice(i * B + e0, 16)] = cols[0]
            return 0

        lax.fori_loop(0, B // 16, group, 0)

    start(0, s0, t0, sem0)

    def outer(k, _):
        i0 = 2 * k
        start(i0 + 1, s1, t1, sem1)
        drain(s0, t0, sem0)
        compute(i0, s0, t0)

        @pl.when(i0 + 2 < NCHUNK)
        def _():
            start(i0 + 2, s0, t0, sem0)

        drain(s1, t1, sem1)
        compute(i0 + 1, s1, t1)
        return 0

    lax.fori_loop(0, (NCHUNK - 1) // 2, outer, 0)
    drain(s0, t0, sem0)
    compute(NCHUNK - 1, s0, t0)

    pltpu.sync_copy(out_v, out.at[pl.ds(base, EPW)])


def kernel(h_src, h_dst, edge_label_index, W):
    w = W[0]
    isrc = edge_label_index[0].astype(jnp.int32)
    idst = edge_label_index[1].astype(jnp.int32)
    hsw = _prescale(h_src, w)
    mesh = plsc.VectorSubcoreMesh(
        core_axis_name="c", subcore_axis_name="s", num_cores=NC, num_subcores=NS
    )
    fn = pl.kernel(
        _sc_body,
        out_type=jax.ShapeDtypeStruct((E,), jnp.float32),
        mesh=mesh,
        compiler_params=pltpu.CompilerParams(needs_layout_passes=False),
        scratch_types=[
            pltpu.VMEM((EPW,), jnp.int32),
            pltpu.VMEM((EPW,), jnp.int32),
            pltpu.VMEM((EPW,), jnp.float32),
            pltpu.VMEM((B, D), jnp.float32),
            pltpu.VMEM((B, D), jnp.float32),
            pltpu.VMEM((B, D), jnp.float32),
            pltpu.VMEM((B, D), jnp.float32),
            pltpu.VMEM((256,), jnp.float32),
            pltpu.SemaphoreType.DMA,
            pltpu.SemaphoreType.DMA,
        ],
    )
    return fn(hsw, h_dst, isrc, idst)
